# baseline (speedup 1.0000x reference)
import functools

import jax
import jax.numpy as jnp
from jax import lax
from jax.experimental import pallas as pl
from jax.experimental.pallas import tpu as pltpu
from jax.experimental.pallas import tpu_sc as plsc

N_USERS = 50000
N_ITEMS = 50000
N_NODES = N_USERS + N_ITEMS
D = 16

NC = 2
NS = 16
NW = NC * NS
SUB = 256
NSUB = 3
CHUNK = SUB * NSUB
CH0 = 158
CH1 = 104
G_CHUNKS = NS * (CH0 + CH1)

N_PAD = 100096
ROWS_PER_SUBCORE = N_PAD // NS
CB = 782
N_CCHUNK = N_PAD // CB

_MESH = plsc.VectorSubcoreMesh(core_axis_name="c", subcore_axis_name="s")
_PARAMS = pltpu.CompilerParams(use_tc_tiling_on_sc=False)


@functools.partial(
    pl.kernel,
    out_type=jax.ShapeDtypeStruct((NC, N_PAD, D), jnp.float32),
    mesh=_MESH,
    scratch_types=[
        pltpu.VMEM_SHARED((N_PAD, D), jnp.float32),
        pltpu.VMEM((2, NSUB, SUB), jnp.int32),
        pltpu.VMEM((2, NSUB, SUB), jnp.int32),
        pltpu.VMEM((2, CHUNK // D, D), jnp.float32),
        pltpu.VMEM((2, CHUNK, D), jnp.float32),
        pltpu.SemaphoreType.DMA,
        pltpu.SemaphoreType.DMA,
        pltpu.SemaphoreType.DMA,
    ],
    compiler_params=_PARAMS,
)
def _scatter_layer(x_hbm, zeros_hbm, icol_hbm, irow_hbm, vals_hbm, part_hbm,
                   acc_sh, icol_v, irow_v, vals_v, rows_v, gsem, ssem, isem):
    c = lax.axis_index("c")
    s = lax.axis_index("s")
    my_n = jnp.where(c == 0, CH0, CH1)
    start = jnp.where(c == 0, s * CH0, NS * CH0 + s * CH1)

    def issue_idx(ci, b):
        pltpu.async_copy(icol_hbm.at[start + ci], icol_v.at[b], isem)
        pltpu.async_copy(irow_hbm.at[start + ci], irow_v.at[b], isem)
        pltpu.async_copy(vals_hbm.at[start + ci], vals_v.at[b], isem)

    def wait_idx(b):
        pltpu.make_async_copy(icol_hbm.at[0], icol_v.at[b], isem).wait()
        pltpu.make_async_copy(irow_hbm.at[0], irow_v.at[b], isem).wait()
        pltpu.make_async_copy(vals_hbm.at[0], vals_v.at[b], isem).wait()

    def issue_gathers(b):
        for j in range(NSUB):
            pltpu.async_copy(x_hbm.at[icol_v.at[b, j]],
                             rows_v.at[b, pl.ds(j * SUB, SUB)], gsem)

    def wait_gathers(b):
        for j in range(NSUB):
            pltpu.make_async_copy(x_hbm.at[icol_v.at[b, j]],
                                  rows_v.at[b, pl.ds(j * SUB, SUB)], gsem).wait()

    def drain_scatters(b):
        for j in range(NSUB):
            pltpu.make_async_copy(rows_v.at[b, pl.ds(j * SUB, SUB)],
                                  acc_sh.at[irow_v.at[b, j]], ssem).wait()

    pltpu.sync_copy(zeros_hbm.at[s],
                    acc_sh.at[pl.ds(s * ROWS_PER_SUBCORE, ROWS_PER_SUBCORE)])
    pltpu.sync_copy(icol_hbm.at[start], icol_v.at[0])
    pltpu.sync_copy(irow_hbm.at[start], irow_v.at[0])
    pltpu.sync_copy(vals_hbm.at[start], vals_v.at[0])
    issue_gathers(0)
    issue_idx(1, 1)
    plsc.subcore_barrier()

    def process(ci, b):
        ob = 1 - b
        wait_gathers(b)

        @pl.when(ci + 1 < my_n)
        def _():
            wait_idx(ob)
            issue_gathers(ob)

        for j in range(NSUB):
            @plsc.parallel_loop(j * (SUB // D), (j + 1) * (SUB // D))
            def _(g):
                vv = vals_v[b, g]
                for i in range(D):
                    e = g * D + i
                    rows_v[b, e] = rows_v[b, e] * vv[i]

            pltpu.async_copy(rows_v.at[b, pl.ds(j * SUB, SUB)],
                             acc_sh.at[irow_v.at[b, j]], ssem, add=True)

        drain_scatters(b)

        @pl.when(ci + 2 < my_n)
        def _():
            issue_idx(ci + 2, b)

    @pl.loop(0, my_n, step=2)
    def _(ci2):
        process(ci2, 0)
        process(ci2 + 1, 1)

    plsc.subcore_barrier()
    pltpu.sync_copy(acc_sh.at[pl.ds(s * ROWS_PER_SUBCORE, ROWS_PER_SUBCORE)],
                    part_hbm.at[c, pl.ds(s * ROWS_PER_SUBCORE, ROWS_PER_SUBCORE)])


def _combine_builder(final, scale):
    out_types = (
        jax.ShapeDtypeStruct((N_CCHUNK, CB, D), jnp.float32)
        if final else
        (jax.ShapeDtypeStruct((N_CCHUNK, CB, D), jnp.float32),
         jax.ShapeDtypeStruct((N_CCHUNK, CB, D), jnp.float32))
    )

    @functools.partial(
        pl.kernel,
        out_type=out_types,
        mesh=_MESH,
        scratch_types=[
            pltpu.VMEM((CB, D), jnp.float32),
            pltpu.VMEM((CB, D), jnp.float32),
            pltpu.VMEM((CB, D), jnp.float32),
        ],
        compiler_params=_PARAMS,
    )
    def _combine(part_hbm, accin_hbm, *rest):
        if final:
            accout_hbm, a_v, b_v, c_v = rest
            xnew_hbm = None
        else:
            xnew_hbm, accout_hbm, a_v, b_v, c_v = rest
        c = lax.axis_index("c")
        s = lax.axis_index("s")
        wid = c * NS + s

        def cb_body(k, carry):
            ch = wid + NW * k
            pltpu.sync_copy(part_hbm.at[0, ch], a_v)
            pltpu.sync_copy(part_hbm.at[1, ch], b_v)
            pltpu.sync_copy(accin_hbm.at[ch], c_v)

            def row_body(i, carry2):
                x = a_v[i] + b_v[i]
                if not final:
                    a_v[i] = x
                acc = c_v[i] + x
                c_v[i] = acc * scale if final else acc
                return carry2

            lax.fori_loop(0, CB, row_body, 0)
            if not final:
                pltpu.sync_copy(a_v, xnew_hbm.at[ch])
            pltpu.sync_copy(c_v, accout_hbm.at[ch])
            return carry

        lax.fori_loop(0, N_CCHUNK // NW, cb_body, 0)

    return _combine


def kernel(user_emb, item_emb, adj_indices, adj_values, n_layers):
    n = n_layers if isinstance(n_layers, int) else 3

    x0 = jnp.concatenate([
        user_emb, item_emb,
        jnp.zeros((N_PAD - N_NODES, D), jnp.float32),
    ], axis=0)
    e = adj_values.shape[0]
    e_pad = G_CHUNKS * CHUNK
    pad = e_pad - e
    irow = jnp.concatenate([adj_indices[0], jnp.zeros((pad,), jnp.int32)])
    icol = jnp.concatenate([adj_indices[1], jnp.zeros((pad,), jnp.int32)])
    vals = jnp.concatenate([adj_values, jnp.zeros((pad,), jnp.float32)])
    icol = icol.reshape(G_CHUNKS, NSUB, SUB)
    irow = irow.reshape(G_CHUNKS, NSUB, SUB)
    vals = vals.reshape(G_CHUNKS, CHUNK // D, D)
    zeros = jnp.zeros((NS, ROWS_PER_SUBCORE, D), jnp.float32)

    combine_mid = _combine_builder(False, 1.0)
    combine_fin = _combine_builder(True, 1.0 / (n + 1))

    x = x0
    acc = x0.reshape(N_CCHUNK, CB, D)
    for layer in range(n):
        part = _scatter_layer(x, zeros, icol, irow, vals)
        part_c = part.reshape(NC, N_CCHUNK, CB, D)
        if layer + 1 < n:
            xc, acc = combine_mid(part_c, acc)
            x = xc.reshape(N_PAD, D)
        else:
            out = combine_fin(part_c, acc).reshape(N_PAD, D)

    return (out[:N_USERS], out[N_USERS:N_NODES])


# speedup vs baseline: 1.0087x; 1.0087x over previous
import functools

import jax
import jax.numpy as jnp
from jax import lax
from jax.experimental import pallas as pl
from jax.experimental.pallas import tpu as pltpu
from jax.experimental.pallas import tpu_sc as plsc

N_USERS = 50000
N_ITEMS = 50000
N_NODES = N_USERS + N_ITEMS
D = 16

NC = 2
NS = 16
NW = NC * NS
SUB = 256
NSUB = 3
CHUNK = SUB * NSUB
CH0 = 160
CH1 = 102
G_CHUNKS = NS * (CH0 + CH1)

N_PAD = 100096
ROWS_PER_SUBCORE = N_PAD // NS
CB = 782
N_CCHUNK = N_PAD // CB

_MESH = plsc.VectorSubcoreMesh(core_axis_name="c", subcore_axis_name="s")
_PARAMS = pltpu.CompilerParams(use_tc_tiling_on_sc=False)


@functools.partial(
    pl.kernel,
    out_type=jax.ShapeDtypeStruct((NC, N_PAD, D), jnp.float32),
    mesh=_MESH,
    scratch_types=[
        pltpu.VMEM_SHARED((N_PAD, D), jnp.float32),
        pltpu.VMEM((2, NSUB, SUB), jnp.int32),
        pltpu.VMEM((2, NSUB, SUB), jnp.int32),
        pltpu.VMEM((2, CHUNK // D, D), jnp.float32),
        pltpu.VMEM((2, CHUNK, D), jnp.float32),
        pltpu.SemaphoreType.DMA,
        pltpu.SemaphoreType.DMA,
        pltpu.SemaphoreType.DMA,
    ],
    compiler_params=_PARAMS,
)
def _scatter_layer(x_hbm, zeros_hbm, icol_hbm, irow_hbm, vals_hbm, part_hbm,
                   acc_sh, icol_v, irow_v, vals_v, rows_v, gsem, ssem, isem):
    c = lax.axis_index("c")
    s = lax.axis_index("s")
    my_n = jnp.where(c == 0, CH0, CH1)
    start = jnp.where(c == 0, s * CH0, NS * CH0 + s * CH1)

    def issue_idx(ci, b):
        pltpu.async_copy(icol_hbm.at[start + ci], icol_v.at[b], isem)
        pltpu.async_copy(irow_hbm.at[start + ci], irow_v.at[b], isem)
        pltpu.async_copy(vals_hbm.at[start + ci], vals_v.at[b], isem)

    def wait_idx(b):
        pltpu.make_async_copy(icol_hbm.at[0], icol_v.at[b], isem).wait()
        pltpu.make_async_copy(irow_hbm.at[0], irow_v.at[b], isem).wait()
        pltpu.make_async_copy(vals_hbm.at[0], vals_v.at[b], isem).wait()

    def issue_gathers(b):
        for j in range(NSUB):
            pltpu.async_copy(x_hbm.at[icol_v.at[b, j]],
                             rows_v.at[b, pl.ds(j * SUB, SUB)], gsem)

    def wait_gathers(b):
        for j in range(NSUB):
            pltpu.make_async_copy(x_hbm.at[icol_v.at[b, j]],
                                  rows_v.at[b, pl.ds(j * SUB, SUB)], gsem).wait()

    def drain_scatters(b):
        for j in range(NSUB):
            pltpu.make_async_copy(rows_v.at[b, pl.ds(j * SUB, SUB)],
                                  acc_sh.at[irow_v.at[b, j]], ssem).wait()

    pltpu.sync_copy(zeros_hbm.at[s],
                    acc_sh.at[pl.ds(s * ROWS_PER_SUBCORE, ROWS_PER_SUBCORE)])
    pltpu.sync_copy(icol_hbm.at[start], icol_v.at[0])
    pltpu.sync_copy(irow_hbm.at[start], irow_v.at[0])
    pltpu.sync_copy(vals_hbm.at[start], vals_v.at[0])
    issue_gathers(0)
    issue_idx(1, 1)
    plsc.subcore_barrier()

    def process(ci, b):
        ob = 1 - b
        wait_gathers(b)

        @pl.when(ci + 1 < my_n)
        def _():
            wait_idx(ob)
            issue_gathers(ob)

        for j in range(NSUB):
            @plsc.parallel_loop(j * (SUB // D), (j + 1) * (SUB // D))
            def _(g):
                vv = vals_v[b, g]
                for i in range(D):
                    e = g * D + i
                    rows_v[b, e] = rows_v[b, e] * vv[i]

            pltpu.async_copy(rows_v.at[b, pl.ds(j * SUB, SUB)],
                             acc_sh.at[irow_v.at[b, j]], ssem, add=True)

        drain_scatters(b)

        @pl.when(ci + 2 < my_n)
        def _():
            issue_idx(ci + 2, b)

    @pl.loop(0, my_n, step=2)
    def _(ci2):
        process(ci2, 0)
        process(ci2 + 1, 1)

    plsc.subcore_barrier()
    pltpu.sync_copy(acc_sh.at[pl.ds(s * ROWS_PER_SUBCORE, ROWS_PER_SUBCORE)],
                    part_hbm.at[c, pl.ds(s * ROWS_PER_SUBCORE, ROWS_PER_SUBCORE)])


def _combine_builder(final, scale):
    out_types = (
        jax.ShapeDtypeStruct((N_CCHUNK, CB, D), jnp.float32)
        if final else
        (jax.ShapeDtypeStruct((N_CCHUNK, CB, D), jnp.float32),
         jax.ShapeDtypeStruct((N_CCHUNK, CB, D), jnp.float32))
    )

    @functools.partial(
        pl.kernel,
        out_type=out_types,
        mesh=_MESH,
        scratch_types=[
            pltpu.VMEM((CB, D), jnp.float32),
            pltpu.VMEM((CB, D), jnp.float32),
            pltpu.VMEM((CB, D), jnp.float32),
        ],
        compiler_params=_PARAMS,
    )
    def _combine(part_hbm, accin_hbm, *rest):
        if final:
            accout_hbm, a_v, b_v, c_v = rest
            xnew_hbm = None
        else:
            xnew_hbm, accout_hbm, a_v, b_v, c_v = rest
        c = lax.axis_index("c")
        s = lax.axis_index("s")
        wid = c * NS + s

        def cb_body(k, carry):
            ch = wid + NW * k
            pltpu.sync_copy(part_hbm.at[0, ch], a_v)
            pltpu.sync_copy(part_hbm.at[1, ch], b_v)
            pltpu.sync_copy(accin_hbm.at[ch], c_v)

            def row_body(i, carry2):
                x = a_v[i] + b_v[i]
                if not final:
                    a_v[i] = x
                acc = c_v[i] + x
                c_v[i] = acc * scale if final else acc
                return carry2

            lax.fori_loop(0, CB, row_body, 0)
            if not final:
                pltpu.sync_copy(a_v, xnew_hbm.at[ch])
            pltpu.sync_copy(c_v, accout_hbm.at[ch])
            return carry

        lax.fori_loop(0, N_CCHUNK // NW, cb_body, 0)

    return _combine


def kernel(user_emb, item_emb, adj_indices, adj_values, n_layers):
    n = n_layers if isinstance(n_layers, int) else 3

    x0 = jnp.concatenate([
        user_emb, item_emb,
        jnp.zeros((N_PAD - N_NODES, D), jnp.float32),
    ], axis=0)
    e = adj_values.shape[0]
    e_pad = G_CHUNKS * CHUNK
    pad = e_pad - e
    irow = jnp.concatenate([adj_indices[0], jnp.zeros((pad,), jnp.int32)])
    icol = jnp.concatenate([adj_indices[1], jnp.zeros((pad,), jnp.int32)])
    vals = jnp.concatenate([adj_values, jnp.zeros((pad,), jnp.float32)])
    icol = icol.reshape(G_CHUNKS, NSUB, SUB)
    irow = irow.reshape(G_CHUNKS, NSUB, SUB)
    vals = vals.reshape(G_CHUNKS, CHUNK // D, D)
    zeros = jnp.zeros((NS, ROWS_PER_SUBCORE, D), jnp.float32)

    combine_mid = _combine_builder(False, 1.0)
    combine_fin = _combine_builder(True, 1.0 / (n + 1))

    x = x0
    acc = x0.reshape(N_CCHUNK, CB, D)
    for layer in range(n):
        part = _scatter_layer(x, zeros, icol, irow, vals)
        part_c = part.reshape(NC, N_CCHUNK, CB, D)
        if layer + 1 < n:
            xc, acc = combine_mid(part_c, acc)
            x = xc.reshape(N_PAD, D)
        else:
            out = combine_fin(part_c, acc).reshape(N_PAD, D)

    return (out[:N_USERS], out[N_USERS:N_NODES])
